# baseline (speedup 1.0000x reference)
import functools

import jax
import jax.numpy as jnp
from jax import lax
from jax.experimental import pallas as pl
from jax.experimental.pallas import tpu as pltpu
from jax.experimental.pallas import tpu_sc as plsc

NUM_TYPES = 100
BATCH = 1024
DUP = 2
NUM_HEADS = 16
LENGTH = 20
HEAD_DIM = 64
ROW = DUP * 1 * NUM_HEADS * LENGTH * HEAD_DIM

NC = 2
NS = 16
NW = NC * NS
B_PER_W = BATCH // NW

M = 2
G = B_PER_W // M


def _gather_body(table_hbm, eidx_hbm, out_hbm, eidx_v, rows_v, gsem, psem):
    wid = lax.axis_index("s") * NC + lax.axis_index("c")
    base = wid * B_PER_W
    pltpu.sync_copy(eidx_hbm.at[pl.ds(wid * G * 8, G * 8)], eidx_v)

    def step(g, carry):
        gcp = pltpu.make_async_copy(
            table_hbm.at[eidx_v.at[pl.ds(8 * g, M)]], rows_v, gsem
        )
        gcp.start()
        gcp.wait()
        pcp = pltpu.make_async_copy(
            rows_v, out_hbm.at[pl.ds(base + M * g, M)], psem
        )
        pcp.start()
        pcp.wait()
        return carry

    lax.fori_loop(0, G, step, 0)


@functools.partial(jax.jit, static_argnames=())
def _gather(table, eidx):
    mesh = plsc.VectorSubcoreMesh(core_axis_name="c", subcore_axis_name="s")
    return pl.kernel(
        _gather_body,
        out_type=jax.ShapeDtypeStruct((BATCH, ROW), jnp.float32),
        mesh=mesh,
        scratch_types=[
            pltpu.VMEM((G * 8,), jnp.int32),
            pltpu.VMEM((M, ROW), jnp.float32),
            pltpu.SemaphoreType.DMA,
            pltpu.SemaphoreType.DMA,
        ],
    )(table, eidx)


def kernel(customer_type_batch, prompt_table):
    idx = customer_type_batch.astype(jnp.int32)
    eidx = jnp.zeros((BATCH // M, 8), jnp.int32).at[:, :M].set(
        idx.reshape(BATCH // M, M)).reshape(-1)
    table = prompt_table.reshape(NUM_TYPES, ROW)
    out = _gather(table, eidx)
    return out.reshape(BATCH, DUP, 1, NUM_HEADS, LENGTH, HEAD_DIM)
